# Initial kernel scaffold; baseline (speedup 1.0000x reference)
import jax
import jax.numpy as jnp
from jax.experimental import pallas as pl


def kernel(x, edge_index, batch, Wl, bl, Wr, br, fcmax_W, fcmax_b, fc1_W, fc1_b, fc2_W, fc2_b):
    raise NotImplementedError("write your pallas kernel here")


# speedup vs baseline: 3.0075x; 3.0075x over previous
import functools

import jax
import jax.numpy as jnp
from jax import lax
from jax.experimental import pallas as pl
from jax.experimental.pallas import tpu as pltpu
from jax.experimental.pallas import tpu_sc as plsc

NN = 10000
EE = 320000
DD = 128
GG = 64
LL = 4
TT = 10

NPT = 313
NPAD = 32 * NPT
KCH = 128
EPAD = EE + 6 * KCH
RCH = 64
NEG = -3.0e38

_mesh = plsc.VectorSubcoreMesh(core_axis_name="c", subcore_axis_name="s")


def _iota16():
    return lax.broadcasted_iota(jnp.int32, (16,), 0)


def _lane(vec, j):
    return lax.squeeze(lax.slice(vec, (j,), (j + 1,)), (0,))


def _splat(val):
    return jnp.full((16,), val, jnp.float32)


@functools.partial(
    pl.kernel,
    mesh=_mesh,
    out_type=jax.ShapeDtypeStruct((NPAD * DD,), jnp.float32),
    scratch_types=[
        pltpu.VMEM_SHARED((48,), jnp.int32),
        pltpu.SMEM((48,), jnp.int32),
        pltpu.VMEM((KCH,), jnp.int32),
        pltpu.VMEM((KCH,), jnp.int32),
        pltpu.VMEM((KCH, DD), jnp.float32),
        pltpu.VMEM((KCH, DD), jnp.float32),
        pltpu.VMEM((KCH,), jnp.int32),
        pltpu.VMEM((KCH,), jnp.int32),
        pltpu.VMEM(((NPT + 1) * DD,), jnp.float32),
        pltpu.SemaphoreType.DMA,
        pltpu.SemaphoreType.DMA,
        pltpu.SemaphoreType.DMA,
        pltpu.SemaphoreType.DMA,
        pltpu.SemaphoreType.DMA,
        pltpu.SemaphoreType.DMA,
    ],
)
def _segmax(h_hbm, src_hbm, dst_hbm, bnd_hbm, agg_hbm,
            bnd_stage, bnd_v, idx0, idx1, row0, row1, dv0, dv1, outb,
            g0, g1, i0, i1, d0, d1):
    t = lax.axis_index("s") * 2 + lax.axis_index("c")
    n0 = t * NPT

    @pl.when(lax.axis_index("s") == 0)
    def _():
        pltpu.sync_copy(bnd_hbm, bnd_stage)

    plsc.subcore_barrier()
    pltpu.sync_copy(bnd_stage, bnd_v)
    e0 = (bnd_v[t] >> 4) << 4
    e1 = (bnd_v[t + 1] >> 4) << 4

    zeros = _splat(0.0)

    def _zrow(i, _):
        outb[pl.ds(i * 16, 16)] = zeros
        return 0

    lax.fori_loop(0, (NPT + 1) * DD // 16, _zrow, 0)

    idx_bufs = (idx0, idx1)
    row_bufs = (row0, row1)
    dst_bufs = (dv0, dv1)
    gsems = (g0, g1)
    isems = (i0, i1)
    dsems = (d0, d1)

    def _fetch(c, b):
        base = pl.multiple_of(e0 + c * KCH, 16)
        pltpu.async_copy(src_hbm.at[pl.ds(base, KCH)], idx_bufs[b], isems[b])
        pltpu.async_copy(dst_hbm.at[pl.ds(base, KCH)], dst_bufs[b], dsems[b])

    def _wait_fetch(b):
        pltpu.make_async_copy(src_hbm.at[pl.ds(0, KCH)], idx_bufs[b], isems[b]).wait()
        pltpu.make_async_copy(dst_hbm.at[pl.ds(0, KCH)], dst_bufs[b], dsems[b]).wait()

    def _gather(b):
        pltpu.async_copy(h_hbm.at[idx_bufs[b]], row_bufs[b], gsems[b])

    def _wait_gather(b):
        pltpu.make_async_copy(h_hbm.at[pl.ds(0, KCH)], row_bufs[b], gsems[b]).wait()

    _fetch(0, 0)
    _fetch(1, 1)
    _wait_fetch(0)
    _gather(0)

    nch = (e1 - e0 + KCH - 1) // KCH
    nit = (nch + 1) // 2

    def _compute(c, b, carry):
        prev_d, acc = carry
        base = e0 + c * KCH
        ngr = jnp.clip((e1 - base) // 16, 0, KCH // 16)
        rbuf = row_bufs[b]
        dbuf = dst_bufs[b]

        def _group(gidx, carry):
            prev_d, acc = carry
            dstv = dbuf[pl.ds(gidx * 16, 16)] - n0
            dstv = jnp.where(dstv < 0, NPT, dstv)
            for j in range(16):
                d = _lane(dstv, j)
                e = gidx * 16 + j
                reset = d != prev_d
                nacc = []
                for f in range(8):
                    row = rbuf[e, pl.ds(f * 16, 16)]
                    a = jnp.where(reset, _splat(NEG), acc[f])
                    a = jnp.maximum(a, row)
                    outb[pl.ds(d * DD + f * 16, 16)] = a
                    nacc.append(a)
                acc = tuple(nacc)
                prev_d = d
            return prev_d, acc

        return lax.fori_loop(0, ngr, _group, (prev_d, acc))

    acc0 = tuple(_splat(NEG) for _ in range(8))

    def _body(i, carry):
        for b in range(2):
            c = 2 * i + b
            _wait_gather(b)
            _wait_fetch(1 - b)
            _gather(1 - b)
            carry = _compute(c, b, carry)
            _fetch(c + 2, b)
        return carry

    lax.fori_loop(0, nit, _body, (jnp.int32(-1), acc0))

    _wait_gather(0)
    _wait_fetch(1)

    pltpu.sync_copy(outb.at[pl.ds(0, NPT * DD)],
                    agg_hbm.at[pl.ds(pl.multiple_of(n0 * DD, 8), NPT * DD)])


@functools.partial(
    pl.kernel,
    mesh=_mesh,
    out_type=jax.ShapeDtypeStruct((GG * LL * DD,), jnp.float32),
    scratch_types=[
        pltpu.VMEM_SHARED((80,), jnp.int32),
        pltpu.SMEM((80,), jnp.int32),
        pltpu.VMEM((RCH, DD), jnp.float32),
        pltpu.VMEM((RCH, DD), jnp.float32),
        pltpu.VMEM((RCH, DD), jnp.float32),
        pltpu.VMEM((RCH, DD), jnp.float32),
        pltpu.VMEM((2 * LL * DD,), jnp.float32),
    ],
)
def _pool(h0, h1, h2, h3, goff_hbm, out_hbm, goff_stage, goff_v,
          rb0, rb1, rb2, rb3, pbuf):
    t = lax.axis_index("s") * 2 + lax.axis_index("c")

    @pl.when(lax.axis_index("s") == 0)
    def _():
        pltpu.sync_copy(goff_hbm, goff_stage)

    plsc.subcore_barrier()
    pltpu.sync_copy(goff_stage, goff_v)
    hs = (h0, h1, h2, h3)
    rbs = (rb0, rb1, rb2, rb3)

    for k in range(2):
        r0 = goff_v[2 * t + k]
        r1 = goff_v[2 * t + k + 1]
        a0 = (r0 >> 3) << 3
        nch = (r1 - a0 + RCH - 1) // RCH

        def _chunk(ch, acc):
            b = pl.multiple_of(a0 + ch * RCH, 8)
            for li in range(LL):
                pltpu.sync_copy(hs[li].at[pl.ds(b, RCH)], rbs[li])
            jlo = jnp.clip(r0 - b, 0, RCH)
            jhi = jnp.clip(r1 - b, 0, RCH)

            def _row(j, acc):
                nacc = []
                for li in range(LL):
                    for f in range(8):
                        nacc.append(jnp.maximum(acc[li * 8 + f],
                                                rbs[li][j, pl.ds(f * 16, 16)]))
                return tuple(nacc)

            return lax.fori_loop(jlo, jhi, _row, acc)

        acc = lax.fori_loop(0, nch, _chunk,
                            tuple(_splat(NEG) for _ in range(LL * 8)))
        for v in range(LL * 8):
            val = jnp.where(acc[v] > -1.0e37, acc[v], 0.0)
            pbuf[pl.ds(k * LL * DD + v * 16, 16)] = val

    pltpu.sync_copy(
        pbuf,
        out_hbm.at[pl.ds(pl.multiple_of(t * 2 * LL * DD, 8), 2 * LL * DD)])


def _dense_body(agg_ref, h_ref, wl_ref, wr_ref, b1_ref, w2_ref, b2_ref, o_ref):
    tmp = (jnp.dot(agg_ref[...], wl_ref[...], preferred_element_type=jnp.float32)
           + jnp.dot(h_ref[...], wr_ref[...], preferred_element_type=jnp.float32)
           + b1_ref[...])
    out = jnp.dot(tmp, w2_ref[...], preferred_element_type=jnp.float32) + b2_ref[...]
    o_ref[...] = jnp.maximum(out, 0.0)


_BLK = NPAD // 4


def _dense(agg, h, wl, wr, b1, w2, b2):
    return pl.pallas_call(
        _dense_body,
        grid=(4,),
        in_specs=[
            pl.BlockSpec((_BLK, DD), lambda i: (i, 0)),
            pl.BlockSpec((_BLK, DD), lambda i: (i, 0)),
            pl.BlockSpec((DD, DD), lambda i: (0, 0)),
            pl.BlockSpec((DD, DD), lambda i: (0, 0)),
            pl.BlockSpec((1, DD), lambda i: (0, 0)),
            pl.BlockSpec((DD, DD), lambda i: (0, 0)),
            pl.BlockSpec((1, DD), lambda i: (0, 0)),
        ],
        out_specs=pl.BlockSpec((_BLK, DD), lambda i: (i, 0)),
        out_shape=jax.ShapeDtypeStruct((NPAD, DD), jnp.float32),
    )(agg, h, wl, wr, b1, w2, b2)


def _head_body(p_ref, w1_ref, b1_ref, w2_ref, b2_ref, lsm_ref, out_ref, ll_ref):
    ll = jnp.maximum(
        jnp.dot(p_ref[...], w1_ref[...], preferred_element_type=jnp.float32)
        + b1_ref[...], 0.0)
    out = jnp.dot(ll, w2_ref[...], preferred_element_type=jnp.float32) + b2_ref[...]
    m = jnp.max(out, axis=1, keepdims=True)
    lse = jnp.log(jnp.sum(jnp.exp(out - m), axis=1, keepdims=True)) + m
    ll_ref[...] = ll
    out_ref[...] = out
    lsm_ref[...] = out - lse


def _head(pooled, w1, b1, w2, b2):
    return pl.pallas_call(
        _head_body,
        out_shape=(
            jax.ShapeDtypeStruct((GG, TT), jnp.float32),
            jax.ShapeDtypeStruct((GG, TT), jnp.float32),
            jax.ShapeDtypeStruct((GG, DD), jnp.float32),
        ),
    )(pooled, w1, b1, w2, b2)


def kernel(x, edge_index, batch, Wl, bl, Wr, br, fcmax_W, fcmax_b,
           fc1_W, fc1_b, fc2_W, fc2_b):
    src = edge_index[0]
    dst = edge_index[1]
    perm = jnp.argsort(dst)
    ssrc = src[perm].astype(jnp.int32)
    sdst = dst[perm].astype(jnp.int32)
    ssrc_p = jnp.concatenate([ssrc, jnp.zeros((EPAD - EE,), jnp.int32)])
    sdst_p = jnp.concatenate([sdst, jnp.zeros((EPAD - EE,), jnp.int32)])

    bnd = jnp.searchsorted(sdst, jnp.arange(33, dtype=jnp.int32) * NPT
                           ).astype(jnp.int32)
    bnd = jnp.concatenate([bnd, jnp.full((15,), EE, jnp.int32)])
    goff = jnp.searchsorted(batch, jnp.arange(65, dtype=jnp.int32)
                            ).astype(jnp.int32)
    goff = jnp.concatenate([goff, jnp.full((15,), NN, jnp.int32)])

    h = jnp.concatenate([x, jnp.zeros((NPAD - NN, DD), x.dtype)])
    outs = []
    for i in range(LL):
        agg = _segmax(h, ssrc_p, sdst_p, bnd).reshape(NPAD, DD)
        h = _dense(agg, h, Wl[i], Wr[i], (bl[i] + br[i]).reshape(1, DD),
                   fcmax_W, fcmax_b.reshape(1, DD))
        outs.append(h)

    pooled = _pool(outs[0], outs[1], outs[2], outs[3], goff
                   ).reshape(GG, LL * DD)
    lsm, out, ll = _head(pooled, fc1_W, fc1_b.reshape(1, DD),
                         fc2_W, fc2_b.reshape(1, TT))
    return (lsm, out, ll)


# speedup vs baseline: 4.9431x; 1.6436x over previous
import functools

import jax
import jax.numpy as jnp
from jax import lax
from jax.experimental import pallas as pl
from jax.experimental.pallas import tpu as pltpu
from jax.experimental.pallas import tpu_sc as plsc

NN = 10000
EE = 320000
DD = 128
GG = 64
LL = 4
TT = 10

NPT = 313
NPAD = 32 * NPT
KCH = 128
DK = 2 * KCH
EPAD = EE + 12 * KCH
RCH = 64
NEG = -3.0e38

_mesh = plsc.VectorSubcoreMesh(core_axis_name="c", subcore_axis_name="s")


def _iota16():
    return lax.broadcasted_iota(jnp.int32, (16,), 0)


def _lane(vec, j):
    return lax.squeeze(lax.slice(vec, (j,), (j + 1,)), (0,))


def _splat(val):
    return jnp.full((16,), val, jnp.float32)


@functools.partial(
    pl.kernel,
    mesh=_mesh,
    out_type=jax.ShapeDtypeStruct((NPAD * DD,), jnp.float32),
    scratch_types=[
        pltpu.VMEM_SHARED((48,), jnp.int32),
        pltpu.SMEM((48,), jnp.int32),
        pltpu.VMEM_SHARED((16 * 4 * DK,), jnp.int32),
        pltpu.SMEM((4 * DK,), jnp.int32),
        pltpu.VMEM((KCH,), jnp.int32),
        pltpu.VMEM((KCH,), jnp.int32),
        pltpu.VMEM((KCH,), jnp.int32),
        pltpu.VMEM((KCH,), jnp.int32),
        pltpu.VMEM((KCH, DD), jnp.float32),
        pltpu.VMEM((KCH, DD), jnp.float32),
        pltpu.VMEM((KCH, DD), jnp.float32),
        pltpu.VMEM((KCH, DD), jnp.float32),
        pltpu.VMEM((NPT * DD,), jnp.float32),
        pltpu.SemaphoreType.DMA,
        pltpu.SemaphoreType.DMA,
        pltpu.SemaphoreType.DMA,
        pltpu.SemaphoreType.DMA,
        pltpu.SemaphoreType.DMA,
        pltpu.SemaphoreType.DMA,
        pltpu.SemaphoreType.DMA,
        pltpu.SemaphoreType.DMA,
        pltpu.SemaphoreType.DMA,
        pltpu.SemaphoreType.DMA,
        pltpu.SemaphoreType.DMA,
        pltpu.SemaphoreType.DMA,
    ],
)
def _segmax(h_hbm, src_hbm, dst_hbm, bnd_hbm, agg_hbm,
            bnd_stage, bnd_s, dst_stage, dst_s,
            ix0, ix1, ix2, ix3, rw0, rw1, rw2, rw3, outb,
            g0, g1, g2, g3, i0, i1, i2, i3, d0, d1, d2, d3):
    t = lax.axis_index("s") * 2 + lax.axis_index("c")
    sid = lax.axis_index("s")
    n0 = t * NPT

    @pl.when(sid == 0)
    def _():
        pltpu.sync_copy(bnd_hbm, bnd_stage)

    plsc.subcore_barrier()
    pltpu.sync_copy(bnd_stage, bnd_s)
    e0 = bnd_s[t]
    e1 = bnd_s[t + 1]
    a0 = (e0 >> 7) << 7

    zeros = _splat(0.0)

    def _zrow(i, _):
        outb[pl.ds(i * 16, 16)] = zeros
        return 0

    lax.fori_loop(0, NPT * DD // 16, _zrow, 0)

    idx_bufs = (ix0, ix1, ix2, ix3)
    row_bufs = (rw0, rw1, rw2, rw3)
    gsems = (g0, g1, g2, g3)
    isems = (i0, i1, i2, i3)
    dsems = (d0, d1, d2, d3)

    def _dslice(b):
        return dst_stage.at[pl.ds(pl.multiple_of((sid * 4 + b) * DK, 8), DK)]

    def _fetch(c, b):
        base = pl.multiple_of(a0 + c * KCH, 128)
        pltpu.async_copy(src_hbm.at[pl.ds(base, KCH)], idx_bufs[b], isems[b])
        pltpu.async_copy(dst_hbm.at[pl.ds(base, DK)], _dslice(b), dsems[b])

    def _wait_fetch(b):
        pltpu.make_async_copy(src_hbm.at[pl.ds(0, KCH)], idx_bufs[b], isems[b]).wait()

    def _wait_dst(b):
        pltpu.make_async_copy(dst_hbm.at[pl.ds(0, DK)], _dslice(b), dsems[b]).wait()

    def _gather(b):
        pltpu.async_copy(h_hbm.at[idx_bufs[b]], row_bufs[b], gsems[b])

    def _wait_gather(b):
        pltpu.make_async_copy(h_hbm.at[pl.ds(0, KCH)], row_bufs[b], gsems[b]).wait()

    for b in range(4):
        _fetch(b, b)
    for b in range(3):
        _wait_fetch(b)
        _gather(b)

    nch = (e1 - a0 + KCH - 1) // KCH
    nit = (nch + 3) // 4

    acc0 = tuple(_splat(NEG) for _ in range(8))

    def _compute(c, b, carry):
        dcur, acc = carry
        base = a0 + c * KCH
        jlo = jnp.clip(e0 - base, 0, KCH)
        jhi = jnp.clip(e1 - base, 0, KCH)
        rbuf = row_bufs[b]
        _wait_dst(b)
        pltpu.sync_copy(_dslice(b), dst_s.at[pl.ds(b * DK, DK)])
        dcur = jnp.where(dcur == NN, dst_s[b * DK + jlo], dcur)

        def _e(j, carry):
            dcur, acc = carry
            dnxt = dst_s[b * DK + j + 1]
            acc = tuple(jnp.maximum(acc[f], rbuf[j, pl.ds(f * 16, 16)])
                        for f in range(8))
            bnd = dcur != dnxt

            @pl.when(bnd)
            def _():
                off = (dcur - n0) * DD
                for f in range(8):
                    outb[pl.ds(off + f * 16, 16)] = acc[f]

            acc = tuple(jnp.where(bnd, _splat(NEG), acc[f]) for f in range(8))
            return (dnxt, acc)

        return lax.fori_loop(jlo, jhi, _e, (dcur, acc))

    def _body(i, carry):
        for b in range(4):
            c = 4 * i + b
            _wait_gather(b)
            _wait_fetch((b + 3) % 4)
            _gather((b + 3) % 4)
            carry = _compute(c, b, carry)
            _fetch(c + 4, b)
        return carry

    lax.fori_loop(0, nit, _body, (jnp.int32(NN), acc0))

    for b in range(3):
        _wait_gather(b)
    _wait_fetch(3)
    for b in range(4):
        _wait_dst(b)

    pltpu.sync_copy(outb,
                    agg_hbm.at[pl.ds(pl.multiple_of(n0 * DD, 8), NPT * DD)])


@functools.partial(
    pl.kernel,
    mesh=_mesh,
    out_type=jax.ShapeDtypeStruct((GG * LL * DD,), jnp.float32),
    scratch_types=[
        pltpu.VMEM_SHARED((80,), jnp.int32),
        pltpu.SMEM((80,), jnp.int32),
        pltpu.VMEM((RCH, DD), jnp.float32),
        pltpu.VMEM((RCH, DD), jnp.float32),
        pltpu.VMEM((RCH, DD), jnp.float32),
        pltpu.VMEM((RCH, DD), jnp.float32),
        pltpu.VMEM((2 * LL * DD,), jnp.float32),
    ],
)
def _pool(h0, h1, h2, h3, goff_hbm, out_hbm, goff_stage, goff_v,
          rb0, rb1, rb2, rb3, pbuf):
    t = lax.axis_index("s") * 2 + lax.axis_index("c")

    @pl.when(lax.axis_index("s") == 0)
    def _():
        pltpu.sync_copy(goff_hbm, goff_stage)

    plsc.subcore_barrier()
    pltpu.sync_copy(goff_stage, goff_v)
    hs = (h0, h1, h2, h3)
    rbs = (rb0, rb1, rb2, rb3)

    for k in range(2):
        r0 = goff_v[2 * t + k]
        r1 = goff_v[2 * t + k + 1]
        a0 = (r0 >> 3) << 3
        nch = (r1 - a0 + RCH - 1) // RCH

        def _chunk(ch, acc):
            b = pl.multiple_of(a0 + ch * RCH, 8)
            for li in range(LL):
                pltpu.sync_copy(hs[li].at[pl.ds(b, RCH)], rbs[li])
            jlo = jnp.clip(r0 - b, 0, RCH)
            jhi = jnp.clip(r1 - b, 0, RCH)

            def _row(j, acc):
                nacc = []
                for li in range(LL):
                    for f in range(8):
                        nacc.append(jnp.maximum(acc[li * 8 + f],
                                                rbs[li][j, pl.ds(f * 16, 16)]))
                return tuple(nacc)

            return lax.fori_loop(jlo, jhi, _row, acc)

        acc = lax.fori_loop(0, nch, _chunk,
                            tuple(_splat(NEG) for _ in range(LL * 8)))
        for v in range(LL * 8):
            val = jnp.where(acc[v] > -1.0e37, acc[v], 0.0)
            pbuf[pl.ds(k * LL * DD + v * 16, 16)] = val

    pltpu.sync_copy(
        pbuf,
        out_hbm.at[pl.ds(pl.multiple_of(t * 2 * LL * DD, 8), 2 * LL * DD)])


def _dense_body(agg_ref, h_ref, wl_ref, wr_ref, b1_ref, w2_ref, b2_ref, o_ref):
    tmp = (jnp.dot(agg_ref[...], wl_ref[...], preferred_element_type=jnp.float32)
           + jnp.dot(h_ref[...], wr_ref[...], preferred_element_type=jnp.float32)
           + b1_ref[...])
    out = jnp.dot(tmp, w2_ref[...], preferred_element_type=jnp.float32) + b2_ref[...]
    o_ref[...] = jnp.maximum(out, 0.0)


_BLK = NPAD // 4


def _dense(agg, h, wl, wr, b1, w2, b2):
    return pl.pallas_call(
        _dense_body,
        grid=(4,),
        in_specs=[
            pl.BlockSpec((_BLK, DD), lambda i: (i, 0)),
            pl.BlockSpec((_BLK, DD), lambda i: (i, 0)),
            pl.BlockSpec((DD, DD), lambda i: (0, 0)),
            pl.BlockSpec((DD, DD), lambda i: (0, 0)),
            pl.BlockSpec((1, DD), lambda i: (0, 0)),
            pl.BlockSpec((DD, DD), lambda i: (0, 0)),
            pl.BlockSpec((1, DD), lambda i: (0, 0)),
        ],
        out_specs=pl.BlockSpec((_BLK, DD), lambda i: (i, 0)),
        out_shape=jax.ShapeDtypeStruct((NPAD, DD), jnp.float32),
    )(agg, h, wl, wr, b1, w2, b2)


def _head_body(p_ref, w1_ref, b1_ref, w2_ref, b2_ref, lsm_ref, out_ref, ll_ref):
    ll = jnp.maximum(
        jnp.dot(p_ref[...], w1_ref[...], preferred_element_type=jnp.float32)
        + b1_ref[...], 0.0)
    out = jnp.dot(ll, w2_ref[...], preferred_element_type=jnp.float32) + b2_ref[...]
    m = jnp.max(out, axis=1, keepdims=True)
    lse = jnp.log(jnp.sum(jnp.exp(out - m), axis=1, keepdims=True)) + m
    ll_ref[...] = ll
    out_ref[...] = out
    lsm_ref[...] = out - lse


def _head(pooled, w1, b1, w2, b2):
    return pl.pallas_call(
        _head_body,
        out_shape=(
            jax.ShapeDtypeStruct((GG, TT), jnp.float32),
            jax.ShapeDtypeStruct((GG, TT), jnp.float32),
            jax.ShapeDtypeStruct((GG, DD), jnp.float32),
        ),
    )(pooled, w1, b1, w2, b2)


def kernel(x, edge_index, batch, Wl, bl, Wr, br, fcmax_W, fcmax_b,
           fc1_W, fc1_b, fc2_W, fc2_b):
    src = edge_index[0]
    dst = edge_index[1]
    perm = jnp.argsort(dst)
    ssrc = src[perm].astype(jnp.int32)
    sdst = dst[perm].astype(jnp.int32)
    ssrc_p = jnp.concatenate([ssrc, jnp.zeros((EPAD - EE,), jnp.int32)])
    sdst_p = jnp.concatenate([sdst, jnp.full((EPAD - EE,), NN, jnp.int32)])

    bnd = jnp.searchsorted(sdst, jnp.arange(33, dtype=jnp.int32) * NPT
                           ).astype(jnp.int32)
    bnd = jnp.concatenate([bnd, jnp.full((15,), EE, jnp.int32)])

    goff = jnp.searchsorted(batch, jnp.arange(65, dtype=jnp.int32)
                            ).astype(jnp.int32)
    goff = jnp.concatenate([goff, jnp.full((15,), NN, jnp.int32)])

    h = jnp.concatenate([x, jnp.zeros((NPAD - NN, DD), x.dtype)])
    outs = []
    for i in range(LL):
        agg = _segmax(h, ssrc_p, sdst_p, bnd).reshape(NPAD, DD)
        h = _dense(agg, h, Wl[i], Wr[i], (bl[i] + br[i]).reshape(1, DD),
                   fcmax_W, fcmax_b.reshape(1, DD))
        outs.append(h)

    pooled = _pool(outs[0], outs[1], outs[2], outs[3], goff
                   ).reshape(GG, LL * DD)
    lsm, out, ll = _head(pooled, fc1_W, fc1_b.reshape(1, DD),
                         fc2_W, fc2_b.reshape(1, TT))
    return (lsm, out, ll)


# speedup vs baseline: 5.2358x; 1.0592x over previous
import functools

import jax
import jax.numpy as jnp
from jax import lax
from jax.experimental import pallas as pl
from jax.experimental.pallas import tpu as pltpu
from jax.experimental.pallas import tpu_sc as plsc

NN = 10000
EE = 320000
DD = 128
GG = 64
LL = 4
TT = 10

NPT = 313
NPAD = 32 * NPT
KCH = 128
DK = 2 * KCH
EPAD = EE + 12 * KCH
RCH = 64
NEG = -3.0e38

_mesh = plsc.VectorSubcoreMesh(core_axis_name="c", subcore_axis_name="s")


def _iota16():
    return lax.broadcasted_iota(jnp.int32, (16,), 0)


def _lane(vec, j):
    return lax.squeeze(lax.slice(vec, (j,), (j + 1,)), (0,))


def _splat(val):
    return jnp.full((16,), val, jnp.float32)


@functools.partial(
    pl.kernel,
    mesh=_mesh,
    out_type=jax.ShapeDtypeStruct((NPAD * DD,), jnp.float32),
    scratch_types=[
        pltpu.VMEM_SHARED((48,), jnp.int32),
        pltpu.SMEM((48,), jnp.int32),
        pltpu.VMEM_SHARED((16 * 4 * DK,), jnp.int32),
        pltpu.SMEM((4 * DK,), jnp.int32),
        pltpu.VMEM((KCH,), jnp.int32),
        pltpu.VMEM((KCH,), jnp.int32),
        pltpu.VMEM((KCH,), jnp.int32),
        pltpu.VMEM((KCH,), jnp.int32),
        pltpu.VMEM((KCH, DD), jnp.float32),
        pltpu.VMEM((KCH, DD), jnp.float32),
        pltpu.VMEM((KCH, DD), jnp.float32),
        pltpu.VMEM((KCH, DD), jnp.float32),
        pltpu.VMEM((NPT * DD,), jnp.float32),
        pltpu.SemaphoreType.DMA,
        pltpu.SemaphoreType.DMA,
        pltpu.SemaphoreType.DMA,
        pltpu.SemaphoreType.DMA,
        pltpu.SemaphoreType.DMA,
        pltpu.SemaphoreType.DMA,
        pltpu.SemaphoreType.DMA,
        pltpu.SemaphoreType.DMA,
        pltpu.SemaphoreType.DMA,
        pltpu.SemaphoreType.DMA,
        pltpu.SemaphoreType.DMA,
        pltpu.SemaphoreType.DMA,
    ],
)
def _segmax(h_hbm, src_hbm, dst_hbm, bnd_hbm, agg_hbm,
            bnd_stage, bnd_s, dst_stage, dst_s,
            ix0, ix1, ix2, ix3, rw0, rw1, rw2, rw3, outb,
            g0, g1, g2, g3, i0, i1, i2, i3, d0, d1, d2, d3):
    t = lax.axis_index("s") * 2 + lax.axis_index("c")
    sid = lax.axis_index("s")
    n0 = t * NPT

    @pl.when(sid == 0)
    def _():
        pltpu.sync_copy(bnd_hbm, bnd_stage)

    plsc.subcore_barrier()
    pltpu.sync_copy(bnd_stage, bnd_s)
    e0 = bnd_s[t]
    e1 = bnd_s[t + 1]
    a0 = (e0 >> 7) << 7

    zeros = _splat(0.0)

    def _zrow(i, _):
        outb[pl.ds(i * 16, 16)] = zeros
        return 0

    lax.fori_loop(0, NPT * DD // 16, _zrow, 0)

    idx_bufs = (ix0, ix1, ix2, ix3)
    row_bufs = (rw0, rw1, rw2, rw3)
    gsems = (g0, g1, g2, g3)
    isems = (i0, i1, i2, i3)
    dsems = (d0, d1, d2, d3)

    def _dslice(b):
        return dst_stage.at[pl.ds(pl.multiple_of((sid * 4 + b) * DK, 8), DK)]

    def _fetch(c, b):
        base = pl.multiple_of(a0 + c * KCH, 128)
        pltpu.async_copy(src_hbm.at[pl.ds(base, KCH)], idx_bufs[b], isems[b])
        pltpu.async_copy(dst_hbm.at[pl.ds(base, DK)], _dslice(b), dsems[b])

    def _wait_fetch(b):
        pltpu.make_async_copy(src_hbm.at[pl.ds(0, KCH)], idx_bufs[b], isems[b]).wait()

    def _wait_dst(b):
        pltpu.make_async_copy(dst_hbm.at[pl.ds(0, DK)], _dslice(b), dsems[b]).wait()

    def _gather(b):
        pltpu.async_copy(h_hbm.at[idx_bufs[b]], row_bufs[b], gsems[b])

    def _wait_gather(b):
        pltpu.make_async_copy(h_hbm.at[pl.ds(0, KCH)], row_bufs[b], gsems[b]).wait()

    for b in range(4):
        _fetch(b, b)
    for b in range(3):
        _wait_fetch(b)
        _gather(b)

    nch = (e1 - a0 + KCH - 1) // KCH
    nit = (nch + 3) // 4

    acc0 = tuple(_splat(NEG) for _ in range(8))

    def _compute(c, b, carry):
        dcur, acc = carry
        base = a0 + c * KCH
        jlo = jnp.clip(e0 - base, 0, KCH)
        jhi = jnp.clip(e1 - base, 0, KCH)
        rbuf = row_bufs[b]
        _wait_dst(b)
        pltpu.sync_copy(_dslice(b), dst_s.at[pl.ds(b * DK, DK)])
        dcur = jnp.where(dcur == NN, dst_s[b * DK + jlo], dcur)

        def _e(j, carry):
            dcur, acc = carry
            dnxt = dst_s[b * DK + j + 1]
            acc = tuple(jnp.maximum(acc[f], rbuf[j, pl.ds(f * 16, 16)])
                        for f in range(8))
            bnd = dcur != dnxt

            @pl.when(bnd)
            def _():
                off = (dcur - n0) * DD
                for f in range(8):
                    outb[pl.ds(off + f * 16, 16)] = acc[f]

            acc = tuple(jnp.where(bnd, _splat(NEG), acc[f]) for f in range(8))
            return (dnxt, acc)

        return lax.fori_loop(jlo, jhi, _e, (dcur, acc))

    def _body(i, carry):
        for b in range(4):
            c = 4 * i + b
            _wait_gather(b)
            _wait_fetch((b + 3) % 4)
            _gather((b + 3) % 4)
            carry = _compute(c, b, carry)
            _fetch(c + 4, b)
        return carry

    lax.fori_loop(0, nit, _body, (jnp.int32(NN), acc0))

    for b in range(3):
        _wait_gather(b)
    _wait_fetch(3)
    for b in range(4):
        _wait_dst(b)

    pltpu.sync_copy(outb,
                    agg_hbm.at[pl.ds(pl.multiple_of(n0 * DD, 8), NPT * DD)])


@functools.partial(
    pl.kernel,
    mesh=_mesh,
    out_type=jax.ShapeDtypeStruct((GG * LL * DD,), jnp.float32),
    scratch_types=[
        pltpu.VMEM_SHARED((80,), jnp.int32),
        pltpu.SMEM((80,), jnp.int32),
        pltpu.VMEM((RCH, DD), jnp.float32),
        pltpu.VMEM((RCH, DD), jnp.float32),
        pltpu.VMEM((RCH, DD), jnp.float32),
        pltpu.VMEM((RCH, DD), jnp.float32),
        pltpu.VMEM((2 * LL * DD,), jnp.float32),
    ],
)
def _pool(h0, h1, h2, h3, goff_hbm, out_hbm, goff_stage, goff_v,
          rb0, rb1, rb2, rb3, pbuf):
    t = lax.axis_index("s") * 2 + lax.axis_index("c")

    @pl.when(lax.axis_index("s") == 0)
    def _():
        pltpu.sync_copy(goff_hbm, goff_stage)

    plsc.subcore_barrier()
    pltpu.sync_copy(goff_stage, goff_v)
    hs = (h0, h1, h2, h3)
    rbs = (rb0, rb1, rb2, rb3)

    for k in range(2):
        r0 = goff_v[2 * t + k]
        r1 = goff_v[2 * t + k + 1]
        a0 = (r0 >> 3) << 3
        nch = (r1 - a0 + RCH - 1) // RCH

        def _chunk(ch, acc):
            b = pl.multiple_of(a0 + ch * RCH, 8)
            for li in range(LL):
                pltpu.sync_copy(hs[li].at[pl.ds(b, RCH)], rbs[li])
            jlo = jnp.clip(r0 - b, 0, RCH)
            jhi = jnp.clip(r1 - b, 0, RCH)

            def _row(j, acc):
                nacc = []
                for li in range(LL):
                    for f in range(8):
                        nacc.append(jnp.maximum(acc[li * 8 + f],
                                                rbs[li][j, pl.ds(f * 16, 16)]))
                return tuple(nacc)

            return lax.fori_loop(jlo, jhi, _row, acc)

        acc = lax.fori_loop(0, nch, _chunk,
                            tuple(_splat(NEG) for _ in range(LL * 8)))
        for v in range(LL * 8):
            val = jnp.where(acc[v] > -1.0e37, acc[v], 0.0)
            pbuf[pl.ds(k * LL * DD + v * 16, 16)] = val

    pltpu.sync_copy(
        pbuf,
        out_hbm.at[pl.ds(pl.multiple_of(t * 2 * LL * DD, 8), 2 * LL * DD)])


def _dense_body(agg_ref, h_ref, wl_ref, wr_ref, b1_ref, w2_ref, b2_ref, o_ref):
    tmp = (jnp.dot(agg_ref[...], wl_ref[...], preferred_element_type=jnp.float32)
           + jnp.dot(h_ref[...], wr_ref[...], preferred_element_type=jnp.float32)
           + b1_ref[...])
    out = jnp.dot(tmp, w2_ref[...], preferred_element_type=jnp.float32) + b2_ref[...]
    o_ref[...] = jnp.maximum(out, 0.0)


_BLK = NPAD // 4


def _dense(agg, h, wl, wr, b1, w2, b2):
    return pl.pallas_call(
        _dense_body,
        grid=(4,),
        in_specs=[
            pl.BlockSpec((_BLK, DD), lambda i: (i, 0)),
            pl.BlockSpec((_BLK, DD), lambda i: (i, 0)),
            pl.BlockSpec((DD, DD), lambda i: (0, 0)),
            pl.BlockSpec((DD, DD), lambda i: (0, 0)),
            pl.BlockSpec((1, DD), lambda i: (0, 0)),
            pl.BlockSpec((DD, DD), lambda i: (0, 0)),
            pl.BlockSpec((1, DD), lambda i: (0, 0)),
        ],
        out_specs=pl.BlockSpec((_BLK, DD), lambda i: (i, 0)),
        out_shape=jax.ShapeDtypeStruct((NPAD, DD), jnp.float32),
    )(agg, h, wl, wr, b1, w2, b2)


def _head_body(p_ref, w1_ref, b1_ref, w2_ref, b2_ref, lsm_ref, out_ref, ll_ref):
    ll = jnp.maximum(
        jnp.dot(p_ref[...], w1_ref[...], preferred_element_type=jnp.float32)
        + b1_ref[...], 0.0)
    out = jnp.dot(ll, w2_ref[...], preferred_element_type=jnp.float32) + b2_ref[...]
    m = jnp.max(out, axis=1, keepdims=True)
    lse = jnp.log(jnp.sum(jnp.exp(out - m), axis=1, keepdims=True)) + m
    ll_ref[...] = ll
    out_ref[...] = out
    lsm_ref[...] = out - lse


def _head(pooled, w1, b1, w2, b2):
    return pl.pallas_call(
        _head_body,
        out_shape=(
            jax.ShapeDtypeStruct((GG, TT), jnp.float32),
            jax.ShapeDtypeStruct((GG, TT), jnp.float32),
            jax.ShapeDtypeStruct((GG, DD), jnp.float32),
        ),
    )(pooled, w1, b1, w2, b2)


def kernel(x, edge_index, batch, Wl, bl, Wr, br, fcmax_W, fcmax_b,
           fc1_W, fc1_b, fc2_W, fc2_b):
    src = edge_index[0]
    dst = edge_index[1]
    sdst, ssrc = jax.lax.sort((dst, src), num_keys=1)
    ssrc_p = jnp.concatenate([ssrc, jnp.zeros((EPAD - EE,), jnp.int32)])
    sdst_p = jnp.concatenate([sdst, jnp.full((EPAD - EE,), NN, jnp.int32)])

    bnd = jnp.searchsorted(sdst, jnp.arange(33, dtype=jnp.int32) * NPT
                           ).astype(jnp.int32)
    bnd = jnp.concatenate([bnd, jnp.full((15,), EE, jnp.int32)])

    goff = jnp.searchsorted(batch, jnp.arange(65, dtype=jnp.int32)
                            ).astype(jnp.int32)
    goff = jnp.concatenate([goff, jnp.full((15,), NN, jnp.int32)])

    h = jnp.concatenate([x, jnp.zeros((NPAD - NN, DD), x.dtype)])
    outs = []
    for i in range(LL):
        agg = _segmax(h, ssrc_p, sdst_p, bnd).reshape(NPAD, DD)
        h = _dense(agg, h, Wl[i], Wr[i], (bl[i] + br[i]).reshape(1, DD),
                   fcmax_W, fcmax_b.reshape(1, DD))
        outs.append(h)

    pooled = _pool(outs[0], outs[1], outs[2], outs[3], goff
                   ).reshape(GG, LL * DD)
    lsm, out, ll = _head(pooled, fc1_W, fc1_b.reshape(1, DD),
                         fc2_W, fc2_b.reshape(1, TT))
    return (lsm, out, ll)


# speedup vs baseline: 5.2473x; 1.0022x over previous
import functools

import jax
import jax.numpy as jnp
from jax import lax
from jax.experimental import pallas as pl
from jax.experimental.pallas import tpu as pltpu
from jax.experimental.pallas import tpu_sc as plsc

NN = 10000
EE = 320000
DD = 128
GG = 64
LL = 4
TT = 10

NPT = 314
NPAD = 32 * NPT
KCH = 128
DK = 2 * KCH
EPAD = EE + 12 * KCH
RCH = 64
NEG = -3.0e38

_mesh = plsc.VectorSubcoreMesh(core_axis_name="c", subcore_axis_name="s")


def _iota16():
    return lax.broadcasted_iota(jnp.int32, (16,), 0)


def _lane(vec, j):
    return lax.squeeze(lax.slice(vec, (j,), (j + 1,)), (0,))


def _splat(val):
    return jnp.full((16,), val, jnp.float32)


def _bsplat(val):
    return jnp.full((32,), val, jnp.bfloat16)


DW = DD // 2


@functools.partial(
    pl.kernel,
    mesh=_mesh,
    out_type=jax.ShapeDtypeStruct((NPAD * DD,), jnp.float32),
    scratch_types=[
        pltpu.VMEM_SHARED((48,), jnp.int32),
        pltpu.SMEM((48,), jnp.int32),
        pltpu.VMEM_SHARED((16 * 4 * DK,), jnp.int32),
        pltpu.SMEM((4 * DK,), jnp.int32),
        pltpu.VMEM((KCH,), jnp.int32),
        pltpu.VMEM((KCH,), jnp.int32),
        pltpu.VMEM((KCH,), jnp.int32),
        pltpu.VMEM((KCH,), jnp.int32),
        pltpu.VMEM((KCH, DD), jnp.float32),
        pltpu.VMEM((KCH, DD), jnp.float32),
        pltpu.VMEM((KCH, DD), jnp.float32),
        pltpu.VMEM((KCH, DD), jnp.float32),
        pltpu.VMEM((NPT * DD,), jnp.float32),
        pltpu.SemaphoreType.DMA,
        pltpu.SemaphoreType.DMA,
        pltpu.SemaphoreType.DMA,
        pltpu.SemaphoreType.DMA,
        pltpu.SemaphoreType.DMA,
        pltpu.SemaphoreType.DMA,
        pltpu.SemaphoreType.DMA,
        pltpu.SemaphoreType.DMA,
        pltpu.SemaphoreType.DMA,
        pltpu.SemaphoreType.DMA,
        pltpu.SemaphoreType.DMA,
        pltpu.SemaphoreType.DMA,
    ],
)
def _segmax(h_hbm, src_hbm, dst_hbm, bnd_hbm, agg_hbm,
            bnd_stage, bnd_s, dst_stage, dst_s,
            ix0, ix1, ix2, ix3, rw0, rw1, rw2, rw3, outb,
            g0, g1, g2, g3, i0, i1, i2, i3, d0, d1, d2, d3):
    t = lax.axis_index("s") * 2 + lax.axis_index("c")
    sid = lax.axis_index("s")
    n0 = t * NPT

    @pl.when(sid == 0)
    def _():
        pltpu.sync_copy(bnd_hbm, bnd_stage)

    plsc.subcore_barrier()
    pltpu.sync_copy(bnd_stage, bnd_s)
    e0 = bnd_s[t]
    e1 = bnd_s[t + 1]
    a0 = (e0 >> 7) << 7

    zeros = _splat(0.0)

    def _zrow(i, _):
        outb[pl.ds(i * 16, 16)] = zeros
        return 0

    lax.fori_loop(0, NPT * DD // 16, _zrow, 0)

    idx_bufs = (ix0, ix1, ix2, ix3)
    row_bufs = (rw0, rw1, rw2, rw3)
    gsems = (g0, g1, g2, g3)
    isems = (i0, i1, i2, i3)
    dsems = (d0, d1, d2, d3)

    def _dslice(b):
        return dst_stage.at[pl.ds(pl.multiple_of((sid * 4 + b) * DK, 8), DK)]

    def _fetch(c, b):
        base = pl.multiple_of(a0 + c * KCH, 128)
        pltpu.async_copy(src_hbm.at[pl.ds(base, KCH)], idx_bufs[b], isems[b])
        pltpu.async_copy(dst_hbm.at[pl.ds(base, DK)], _dslice(b), dsems[b])

    def _wait_fetch(b):
        pltpu.make_async_copy(src_hbm.at[pl.ds(0, KCH)], idx_bufs[b], isems[b]).wait()

    def _wait_dst(b):
        pltpu.make_async_copy(dst_hbm.at[pl.ds(0, DK)], _dslice(b), dsems[b]).wait()

    def _gather(b):
        pltpu.async_copy(h_hbm.at[idx_bufs[b]], row_bufs[b], gsems[b])

    def _wait_gather(b):
        pltpu.make_async_copy(h_hbm.at[pl.ds(0, KCH)], row_bufs[b], gsems[b]).wait()

    for b in range(4):
        _fetch(b, b)
    for b in range(3):
        _wait_fetch(b)
        _gather(b)

    nch = (e1 - a0 + KCH - 1) // KCH
    nit = (nch + 3) // 4

    acc0 = tuple(_splat(NEG) for _ in range(8))

    def _compute(c, b, carry):
        dcur, acc = carry
        base = a0 + c * KCH
        jlo = jnp.clip(e0 - base, 0, KCH)
        jhi = jnp.clip(e1 - base, 0, KCH)
        rbuf = row_bufs[b]
        _wait_dst(b)
        pltpu.sync_copy(_dslice(b), dst_s.at[pl.ds(b * DK, DK)])
        dcur = jnp.where(dcur == NN, dst_s[b * DK + jlo], dcur)

        def _e(j, carry):
            dcur, acc = carry
            dnxt = dst_s[b * DK + j + 1]
            acc = tuple(jnp.maximum(acc[f], rbuf[j, pl.ds(f * 16, 16)])
                        for f in range(8))
            bnd = dcur != dnxt

            @pl.when(bnd)
            def _():
                off = (dcur - n0) * DD
                for f in range(8):
                    outb[pl.ds(off + f * 16, 16)] = acc[f]

            acc = tuple(jnp.where(bnd, _splat(NEG), acc[f]) for f in range(8))
            return (dnxt, acc)

        return lax.fori_loop(jlo, jhi, _e, (dcur, acc))

    def _body(i, carry):
        for b in range(4):
            c = 4 * i + b
            _wait_gather(b)
            _wait_fetch((b + 3) % 4)
            _gather((b + 3) % 4)
            carry = _compute(c, b, carry)
            _fetch(c + 4, b)
        return carry

    lax.fori_loop(0, nit, _body, (jnp.int32(NN), acc0))

    for b in range(3):
        _wait_gather(b)
    _wait_fetch(3)
    for b in range(4):
        _wait_dst(b)

    pltpu.sync_copy(outb,
                    agg_hbm.at[pl.ds(pl.multiple_of(n0 * DD, 8), NPT * DD)])


@functools.partial(
    pl.kernel,
    mesh=_mesh,
    out_type=jax.ShapeDtypeStruct((GG * LL * DD,), jnp.float32),
    scratch_types=[
        pltpu.VMEM_SHARED((80,), jnp.int32),
        pltpu.SMEM((80,), jnp.int32),
        pltpu.VMEM((RCH, DD), jnp.float32),
        pltpu.VMEM((RCH, DD), jnp.float32),
        pltpu.VMEM((RCH, DD), jnp.float32),
        pltpu.VMEM((RCH, DD), jnp.float32),
        pltpu.VMEM((2 * LL * DD,), jnp.float32),
    ],
)
def _pool(h0, h1, h2, h3, goff_hbm, out_hbm, goff_stage, goff_v,
          rb0, rb1, rb2, rb3, pbuf):
    t = lax.axis_index("s") * 2 + lax.axis_index("c")

    @pl.when(lax.axis_index("s") == 0)
    def _():
        pltpu.sync_copy(goff_hbm, goff_stage)

    plsc.subcore_barrier()
    pltpu.sync_copy(goff_stage, goff_v)
    hs = (h0, h1, h2, h3)
    rbs = (rb0, rb1, rb2, rb3)

    for k in range(2):
        r0 = goff_v[2 * t + k]
        r1 = goff_v[2 * t + k + 1]
        a0 = (r0 >> 3) << 3
        nch = (r1 - a0 + RCH - 1) // RCH

        def _chunk(ch, acc):
            b = pl.multiple_of(a0 + ch * RCH, 8)
            for li in range(LL):
                pltpu.sync_copy(hs[li].at[pl.ds(b, RCH)], rbs[li])
            jlo = jnp.clip(r0 - b, 0, RCH)
            jhi = jnp.clip(r1 - b, 0, RCH)

            def _row(j, acc):
                nacc = []
                for li in range(LL):
                    for f in range(8):
                        nacc.append(jnp.maximum(acc[li * 8 + f],
                                                rbs[li][j, pl.ds(f * 16, 16)]))
                return tuple(nacc)

            return lax.fori_loop(jlo, jhi, _row, acc)

        acc = lax.fori_loop(0, nch, _chunk,
                            tuple(_splat(NEG) for _ in range(LL * 8)))
        for v in range(LL * 8):
            val = jnp.where(acc[v] > -1.0e37, acc[v], 0.0)
            pbuf[pl.ds(k * LL * DD + v * 16, 16)] = val

    pltpu.sync_copy(
        pbuf,
        out_hbm.at[pl.ds(pl.multiple_of(t * 2 * LL * DD, 8), 2 * LL * DD)])


def _dense_body(agg_ref, h_ref, wl_ref, wr_ref, b1_ref, w2_ref, b2_ref, o_ref):
    agg = agg_ref[...].astype(jnp.float32)
    tmp = (jnp.dot(agg, wl_ref[...], preferred_element_type=jnp.float32)
           + jnp.dot(h_ref[...], wr_ref[...], preferred_element_type=jnp.float32)
           + b1_ref[...])
    out = jnp.dot(tmp, w2_ref[...], preferred_element_type=jnp.float32) + b2_ref[...]
    o_ref[...] = jnp.maximum(out, 0.0)


_BLK = NPAD // 4


def _dense(agg, h, wl, wr, b1, w2, b2):
    return pl.pallas_call(
        _dense_body,
        grid=(4,),
        in_specs=[
            pl.BlockSpec((_BLK, DD), lambda i: (i, 0)),
            pl.BlockSpec((_BLK, DD), lambda i: (i, 0)),
            pl.BlockSpec((DD, DD), lambda i: (0, 0)),
            pl.BlockSpec((DD, DD), lambda i: (0, 0)),
            pl.BlockSpec((1, DD), lambda i: (0, 0)),
            pl.BlockSpec((DD, DD), lambda i: (0, 0)),
            pl.BlockSpec((1, DD), lambda i: (0, 0)),
        ],
        out_specs=pl.BlockSpec((_BLK, DD), lambda i: (i, 0)),
        out_shape=jax.ShapeDtypeStruct((NPAD, DD), jnp.float32),
    )(agg, h, wl, wr, b1, w2, b2)


def _head_body(p_ref, w1_ref, b1_ref, w2_ref, b2_ref, lsm_ref, out_ref, ll_ref):
    ll = jnp.maximum(
        jnp.dot(p_ref[...], w1_ref[...], preferred_element_type=jnp.float32)
        + b1_ref[...], 0.0)
    out = jnp.dot(ll, w2_ref[...], preferred_element_type=jnp.float32) + b2_ref[...]
    m = jnp.max(out, axis=1, keepdims=True)
    lse = jnp.log(jnp.sum(jnp.exp(out - m), axis=1, keepdims=True)) + m
    ll_ref[...] = ll
    out_ref[...] = out
    lsm_ref[...] = out - lse


def _head(pooled, w1, b1, w2, b2):
    return pl.pallas_call(
        _head_body,
        out_shape=(
            jax.ShapeDtypeStruct((GG, TT), jnp.float32),
            jax.ShapeDtypeStruct((GG, TT), jnp.float32),
            jax.ShapeDtypeStruct((GG, DD), jnp.float32),
        ),
    )(pooled, w1, b1, w2, b2)


def kernel(x, edge_index, batch, Wl, bl, Wr, br, fcmax_W, fcmax_b,
           fc1_W, fc1_b, fc2_W, fc2_b):
    src = edge_index[0]
    dst = edge_index[1]
    sdst, ssrc = jax.lax.sort((dst, src), num_keys=1)
    ssrc_p = jnp.concatenate([ssrc, jnp.zeros((EPAD - EE,), jnp.int32)])
    sdst_p = jnp.concatenate([sdst, jnp.full((EPAD - EE,), NN, jnp.int32)])

    bnd = jnp.searchsorted(sdst, jnp.arange(33, dtype=jnp.int32) * NPT
                           ).astype(jnp.int32)
    bnd = jnp.concatenate([bnd, jnp.full((15,), EE, jnp.int32)])

    goff = jnp.searchsorted(batch, jnp.arange(65, dtype=jnp.int32)
                            ).astype(jnp.int32)
    goff = jnp.concatenate([goff, jnp.full((15,), NN, jnp.int32)])

    h = jnp.concatenate([x, jnp.zeros((NPAD - NN, DD), x.dtype)])
    outs = []
    for i in range(LL):
        agg = _segmax(h, ssrc_p, sdst_p, bnd).reshape(NPAD, DD)
        h = _dense(agg, h, Wl[i], Wr[i], (bl[i] + br[i]).reshape(1, DD),
                   fcmax_W, fcmax_b.reshape(1, DD))
        outs.append(h)

    pooled = _pool(outs[0], outs[1], outs[2], outs[3], goff
                   ).reshape(GG, LL * DD)
    lsm, out, ll = _head(pooled, fc1_W, fc1_b.reshape(1, DD),
                         fc2_W, fc2_b.reshape(1, TT))
    return (lsm, out, ll)
